# baseline (speedup 1.0000x reference)
import functools

import jax
import jax.numpy as jnp
from jax import lax
from jax.experimental import pallas as pl
from jax.experimental.pallas import tpu as pltpu
from jax.experimental.pallas import tpu_sc as plsc

LANES = 16
CHUNK = 128
NBUF = 5


@functools.lru_cache(maxsize=None)
def _build(batch: int, seq: int, seq_pad: int, d: int, dpad: int,
           n_workers: int):
    b_per_w = batch // n_workers
    n_jt = b_per_w // CHUNK
    mesh = plsc.VectorSubcoreMesh(core_axis_name="c", subcore_axis_name="s")

    @functools.partial(
        pl.kernel,
        mesh=mesh,
        out_type=jax.ShapeDtypeStruct((seq, d, batch), jnp.float32),
        scratch_types=[
            pltpu.VMEM((seq_pad, CHUNK), jnp.int32),
            pltpu.VMEM((seq_pad, CHUNK), jnp.int32),
            pltpu.VMEM((NBUF, CHUNK, dpad), jnp.float32),
            pltpu.VMEM((d, CHUNK), jnp.float32),
        ] + [pltpu.SemaphoreType.DMA] * NBUF,
        compiler_params=pltpu.CompilerParams(use_tc_tiling_on_sc=True,
                                             needs_layout_passes=False),
    )
    def k(idxt_hbm, w2_hbm, out_hbm, idx_v, ihalf, gbufs, tbuf, *gsems):
        nc = plsc.get_sparse_core_info().num_cores
        wid = lax.axis_index("s") * nc + lax.axis_index("c")
        base = wid * b_per_w
        row_ids = [lax.iota(jnp.int32, LANES) + LANES * g
                   for g in range(CHUNK // LANES)]

        for jt in range(n_jt):
            b0 = base + jt * CHUNK
            pltpu.sync_copy(idxt_hbm.at[:, pl.ds(b0, CHUNK)], idx_v)

            def shift_row(s, c):
                for g in range(CHUNK // LANES):
                    sl = pl.ds(LANES * g, LANES)
                    ihalf[s, sl] = idx_v[s, sl] >> 1
                return c

            lax.fori_loop(0, seq_pad, shift_row, 0)

            for b in range(NBUF):
                pltpu.async_copy(w2_hbm.at[ihalf.at[b]], gbufs.at[b],
                                 gsems[b])

            def step(st, carry):
                for b in range(NBUF):
                    s = st * NBUF + b
                    pltpu.make_async_copy(w2_hbm.at[ihalf.at[s]],
                                          gbufs.at[b], gsems[b]).wait()

                    par64 = [(idx_v[s, pl.ds(LANES * g, LANES)] & 1) * 64
                             for g in range(CHUNK // LANES)]

                    @plsc.parallel_loop(0, d, unroll=8)
                    def _(dd):
                        for g in range(CHUNK // LANES):
                            vec = plsc.load_gather(gbufs.at[b],
                                                   [row_ids[g],
                                                    par64[g] + dd])
                            tbuf[dd, pl.ds(LANES * g, LANES)] = vec
                    pltpu.sync_copy(tbuf, out_hbm.at[s, :, pl.ds(b0, CHUNK)])
                    sn = jnp.minimum(s + NBUF, seq - 1)
                    pltpu.async_copy(w2_hbm.at[ihalf.at[sn]], gbufs.at[b],
                                     gsems[b])
                return carry

            lax.fori_loop(0, seq // NBUF, step, 0)

            for b in range(NBUF):
                pltpu.make_async_copy(w2_hbm.at[ihalf.at[seq - 1]],
                                      gbufs.at[b], gsems[b]).wait()

    return k


def kernel(token_ids, weight):
    batch, seq = token_ids.shape
    vocab, d = weight.shape
    info = plsc.get_sparse_core_info()
    n_workers = info.num_cores * info.num_subcores
    seq_pad = (seq + 7) // 8 * 8
    dpad = 128
    idxt = jnp.pad(token_ids.T.astype(jnp.int32), ((0, seq_pad - seq), (0, 0)))
    w2 = weight.reshape(vocab // 2, 2 * d)
    o = _build(batch, seq, seq_pad, d, dpad, n_workers)(idxt, w2)
    return o.transpose(2, 0, 1)


# speedup vs baseline: 1.0724x; 1.0724x over previous
import functools

import jax
import jax.numpy as jnp
from jax import lax
from jax.experimental import pallas as pl
from jax.experimental.pallas import tpu as pltpu
from jax.experimental.pallas import tpu_sc as plsc

CHUNK = 128
NBUF = 8


@functools.lru_cache(maxsize=None)
def _build(batch: int, seq: int, d: int, n_workers: int):
    b_per_w = batch // n_workers
    n_h = b_per_w // CHUNK
    n_chunks = seq * n_h
    mesh = plsc.VectorSubcoreMesh(core_axis_name="c", subcore_axis_name="s")

    @functools.partial(
        pl.kernel,
        mesh=mesh,
        out_type=jax.ShapeDtypeStruct((batch, seq, d), jnp.float32),
        scratch_types=[
            pltpu.VMEM((seq, b_per_w), jnp.int32),
            pltpu.VMEM((NBUF, CHUNK, d), jnp.float32),
        ] + [pltpu.SemaphoreType.DMA] * NBUF,
        compiler_params=pltpu.CompilerParams(use_tc_tiling_on_sc=False),
    )
    def k(idxt_hbm, weight_hbm, out_hbm, idx_v, rows_v, *gsems):
        nc = plsc.get_sparse_core_info().num_cores
        wid = lax.axis_index("s") * nc + lax.axis_index("c")
        base = wid * b_per_w
        pltpu.sync_copy(idxt_hbm.at[:, pl.ds(base, b_per_w)], idx_v)

        def idx_at(g):
            s, h = g // n_h, g % n_h
            return idx_v.at[s, pl.ds(h * CHUNK, CHUNK)]

        def out_at(g):
            s, h = g // n_h, g % n_h
            return out_hbm.at[pl.ds(base + h * CHUNK, CHUNK), s]

        for b in range(NBUF):
            pltpu.async_copy(weight_hbm.at[idx_at(b)], rows_v.at[b],
                             gsems[b])

        def step(st, carry):
            for b in range(NBUF):
                g = st * NBUF + b
                pltpu.make_async_copy(weight_hbm.at[idx_at(g)],
                                      rows_v.at[b], gsems[b]).wait()
                pltpu.sync_copy(rows_v.at[b], out_at(g))
                gn = jnp.minimum(g + NBUF, n_chunks - 1)
                pltpu.async_copy(weight_hbm.at[idx_at(gn)], rows_v.at[b],
                                 gsems[b])
            return carry

        lax.fori_loop(0, n_chunks // NBUF, step, 0)

        for b in range(NBUF):
            pltpu.make_async_copy(weight_hbm.at[idx_at(n_chunks - 1)],
                                  rows_v.at[b], gsems[b]).wait()

    return k


def kernel(token_ids, weight):
    batch, seq = token_ids.shape
    vocab, d = weight.shape
    info = plsc.get_sparse_core_info()
    n_workers = info.num_cores * info.num_subcores
    idx_t = token_ids.T.astype(jnp.int32)
    return _build(batch, seq, d, n_workers)(idx_t, weight)
